# baseline (speedup 1.0000x reference)
import functools

import jax
import jax.numpy as jnp
from jax import lax
from jax.experimental import pallas as pl
from jax.experimental.pallas import tpu as pltpu
from jax.experimental.pallas import tpu_sc as plsc

N = 10000
E = 160000
R = 8
D = 256
RN = R * N

NC = 2
NS = 16
CH = 128
CHUNKS = E // CH
SP_MAX = -(-CHUNKS // NS)
LOAD_W = SP_MAX * CH
PADDED = 1344 * CH

SPLIT = 5000
ACC = 5120
DUMMY = 5000
DST_BITS = 13
ROWS_PER_TILE = ACC // NS
CNT_BITS = 11
BN = 1000

BF = jnp.bfloat16


def _hrk_body(h_ref, w_ref, o_ref):
    o_ref[0] = jnp.dot(h_ref[...].astype(BF), w_ref[0].astype(BF),
                       preferred_element_type=jnp.float32)


def _hrk(h, W):
    return pl.pallas_call(
        _hrk_body,
        grid=(N // BN, R),
        in_specs=[
            pl.BlockSpec((BN, D), lambda nb, r: (nb, 0)),
            pl.BlockSpec((1, D, D), lambda nb, r: (r, 0, 0)),
        ],
        out_specs=pl.BlockSpec((1, BN, D), lambda nb, r: (r, nb, 0)),
        out_shape=jax.ShapeDtypeStruct((R, N, D), jnp.float32),
    )(h, W)


def _mmk_body(h_ref, s_ref, b_ref, o_ref):
    o_ref[...] = (
        jnp.dot(h_ref[...].astype(BF), s_ref[...].astype(BF),
                preferred_element_type=jnp.float32)
        + b_ref[...]
    )


def _mmk(h, S, b2):
    return pl.pallas_call(
        _mmk_body,
        grid=(N // BN,),
        in_specs=[
            pl.BlockSpec((BN, D), lambda nb: (nb, 0)),
            pl.BlockSpec((D, D), lambda nb: (0, 0)),
            pl.BlockSpec((1, D), lambda nb: (0, 0)),
        ],
        out_specs=pl.BlockSpec((BN, D), lambda nb: (nb, 0)),
        out_shape=jax.ShapeDtypeStruct((N, D), jnp.float32),
    )(h, S, b2)


def _slk_body(agg_ref, slp_ref, s_ref, b_ref, h_ref, sl_ref):
    hb = jnp.maximum(agg_ref[0] + slp_ref[...], 0.0)
    h_ref[...] = hb
    sl_ref[...] = (
        jnp.dot(hb.astype(BF), s_ref[...].astype(BF),
                preferred_element_type=jnp.float32)
        + b_ref[...]
    )


def _slk(agg, slp, S, b2):
    return pl.pallas_call(
        _slk_body,
        grid=(N // BN,),
        in_specs=[
            pl.BlockSpec((1, BN, D), lambda nb: (nb // 5, nb % 5, 0)),
            pl.BlockSpec((BN, D), lambda nb: (nb, 0)),
            pl.BlockSpec((D, D), lambda nb: (0, 0)),
            pl.BlockSpec((1, D), lambda nb: (0, 0)),
        ],
        out_specs=[
            pl.BlockSpec((BN, D), lambda nb: (nb, 0)),
            pl.BlockSpec((BN, D), lambda nb: (nb, 0)),
        ],
        out_shape=[
            jax.ShapeDtypeStruct((N, D), jnp.float32),
            jax.ShapeDtypeStruct((N, D), jnp.float32),
        ],
    )(agg, slp, S, b2)


def _fin_body(agg_ref, slp_ref, o_ref):
    nb = pl.program_id(0)
    hb = jnp.maximum(agg_ref[0] + slp_ref[...], 0.0)
    part = jnp.sum(hb, axis=0, keepdims=True)

    @pl.when(nb == 0)
    def _():
        o_ref[...] = jnp.zeros_like(o_ref)

    o_ref[...] += part


def _fin(agg, slp):
    return pl.pallas_call(
        _fin_body,
        grid=(N // BN,),
        in_specs=[
            pl.BlockSpec((1, BN, D), lambda nb: (nb // 5, nb % 5, 0)),
            pl.BlockSpec((BN, D), lambda nb: (nb, 0)),
        ],
        out_specs=pl.BlockSpec((1, D), lambda nb: (0, 0)),
        out_shape=jax.ShapeDtypeStruct((1, D), jnp.float32),
    )(agg, slp)


def _sc_scatter(hr3, pidx2, cnts, zrows):
    mesh = plsc.VectorSubcoreMesh(core_axis_name="c", subcore_axis_name="s")

    @functools.partial(
        pl.kernel,
        mesh=mesh,
        out_type=jax.ShapeDtypeStruct((NC, ACC, 2, 128), jnp.float32),
        scratch_types=[
            pltpu.VMEM((LOAD_W,), jnp.int32),
            pltpu.VMEM_SHARED((32,), jnp.int32),
            pltpu.SMEM((16,), jnp.int32),
            pltpu.VMEM((CH,), jnp.int32),
            pltpu.VMEM((CH,), jnp.int32),
            pltpu.VMEM((CH, 2, 128), jnp.float32),
            pltpu.VMEM_SHARED((ACC, 2, 128), jnp.float32),
            pltpu.SemaphoreType.DMA,
        ],
    )
    def k(hr_hbm, pidx_hbm, cnt_hbm, z_hbm, out_hbm, pks, cnt_sh, cnt_sm,
          gixb, dstb, rows, acc_s, sem0):
        cid = lax.axis_index("c")
        sid = lax.axis_index("s")
        row0 = sid * ROWS_PER_TILE
        pltpu.sync_copy(z_hbm.at[pl.ds(row0, ROWS_PER_TILE)],
                        acc_s.at[pl.ds(row0, ROWS_PER_TILE)])
        @pl.when(sid == 0)
        def _():
            pltpu.sync_copy(cnt_hbm, cnt_sh)

        plsc.subcore_barrier()
        pltpu.sync_copy(cnt_sh.at[pl.ds(cid * 16, 16)], cnt_sm)
        v = cnt_sm[0]
        lo = v >> CNT_BITS
        nch = v & (2**CNT_BITS - 1)
        sp = (nch + NS - 1) >> 4
        myn = jnp.maximum(jnp.minimum(sp, nch - sid * sp), 0)
        start_w = (lo + sid * sp) * CH
        pltpu.sync_copy(pidx_hbm.at[pl.ds(cid * PADDED + start_w, LOAD_W)],
                        pks)

        def body(g, carry):
            for i in range(CH // 16):
                p = pks[pl.ds(g * CH + i * 16, 16)]
                sl = pl.ds(i * 16, 16)
                gixb[sl] = p >> DST_BITS
                dstb[sl] = p & (2**DST_BITS - 1)
            pltpu.async_copy(hr_hbm.at[gixb], rows, sem0).wait()
            pltpu.sync_copy(rows, acc_s.at[dstb], add=True)
            return carry

        lax.fori_loop(0, myn, body, 0)
        plsc.subcore_barrier()

        @pl.when(cid == 0)
        def _():
            pltpu.sync_copy(acc_s.at[pl.ds(row0, ROWS_PER_TILE)],
                            out_hbm.at[0, pl.ds(row0, ROWS_PER_TILE)])

        @pl.when(cid == 1)
        def _():
            pltpu.sync_copy(acc_s.at[pl.ds(row0, ROWS_PER_TILE)],
                            out_hbm.at[1, pl.ds(row0, ROWS_PER_TILE)])

    return k(hr3, pidx2, cnts, zrows)


def kernel(x, edge_index, edge_type, W0, S0, b0, W1, S1, b1, W2, S2, b2,
           W3, S3, b3):
    src, dst = edge_index[0], edge_index[1]
    gidx = edge_type * N + src

    side = (dst >= SPLIT).astype(jnp.int32)
    k0 = E - jnp.sum(side)
    c0 = jnp.cumsum(1 - side) - 1
    c1 = jnp.cumsum(side) - 1
    pos = jnp.where(side == 0, c0, k0 + c1)
    dla = jnp.where(side == 0, dst, DUMMY)
    dlb = jnp.where(side == 1, dst - SPLIT, DUMMY)
    fill = jnp.full((PADDED,), DUMMY, jnp.int32)
    pidx_a = fill.at[pos].set(gidx * 2**DST_BITS + dla)
    pidx_b = fill.at[pos].set(gidx * 2**DST_BITS + dlb)
    pidx2 = jnp.concatenate([pidx_a, pidx_b])
    p0 = (k0 + CH - 1) // CH
    lo1 = k0 // CH
    v0 = 0 * 2**CNT_BITS + p0
    v1 = lo1 * 2**CNT_BITS + (CHUNKS - lo1)
    cnts = jnp.concatenate([jnp.broadcast_to(v0, (16,)),
                            jnp.broadcast_to(v1, (16,))]).astype(jnp.int32)
    zrows = jnp.zeros((ACC, 2, 128), jnp.float32)

    layers = ((W0, S0, b0), (W1, S1, b1), (W2, S2, b2), (W3, S3, b3))
    h = x
    sl = _mmk(x, S0, b0.reshape(1, D))
    agg = None
    for l in range(4):
        HR = _hrk(h, layers[l][0])
        agg4 = _sc_scatter(HR.reshape(RN, 2, 128), pidx2, cnts, zrows)
        agg = agg4.reshape(NC, ACC, D)
        if l < 3:
            S_next, b_next = layers[l + 1][1], layers[l + 1][2]
            h, sl = _slk(agg, sl, S_next, b_next.reshape(1, D))
    out = _fin(agg, sl)
    return out.reshape(1, 1, D)


# speedup vs baseline: 1.6440x; 1.6440x over previous
import functools

import jax
import jax.numpy as jnp
from jax import lax
from jax.experimental import pallas as pl
from jax.experimental.pallas import tpu as pltpu
from jax.experimental.pallas import tpu_sc as plsc

N = 10000
E = 160000
R = 8
R9 = R + 1
D = 256
H = 128
RN = R * N
R9N = R9 * N

NC = 2
NS = 16
CH = 128

P_SUB = -(-E // (NS * 2 * CH)) * 2 * CH
E_PAD = NS * P_SUB
N_CH = P_SUB // CH
NP = N_CH // 2

ACC = 10112
DST_BITS = 14
ROWS_PER_TILE = ACC // NS
BN = 1000

BF = jnp.bfloat16


def _hrk_body(h_ref, w_ref, o_ref):
    res = jnp.dot(h_ref[...].astype(BF), w_ref[0].astype(BF),
                  preferred_element_type=jnp.float32)
    o_ref[0, 0] = res[:, :H]
    o_ref[1, 0] = res[:, H:]


def _hrk(h, W9):
    return pl.pallas_call(
        _hrk_body,
        grid=(N // BN, R9),
        in_specs=[
            pl.BlockSpec((BN, D), lambda nb, r: (nb, 0)),
            pl.BlockSpec((1, D, D), lambda nb, r: (r, 0, 0)),
        ],
        out_specs=pl.BlockSpec((NC, 1, BN, H), lambda nb, r: (0, r, nb, 0)),
        out_shape=jax.ShapeDtypeStruct((NC, R9, N, H), jnp.float32),
    )(h, W9)


def _prep_body(agg_ref, hr_ref, b_ref, h_ref):
    hcat = (jnp.concatenate([agg_ref[0], agg_ref[1]], axis=1)
            + jnp.concatenate([hr_ref[0, 0], hr_ref[1, 0]], axis=1)
            + b_ref[...])
    h_ref[...] = jnp.maximum(hcat, 0.0)


def _prep(agg, HR, b2):
    return pl.pallas_call(
        _prep_body,
        grid=(N // BN,),
        in_specs=[
            pl.BlockSpec((NC, BN, H), lambda nb: (0, nb, 0)),
            pl.BlockSpec((NC, 1, BN, H), lambda nb: (0, R, nb, 0)),
            pl.BlockSpec((1, D), lambda nb: (0, 0)),
        ],
        out_specs=pl.BlockSpec((BN, D), lambda nb: (nb, 0)),
        out_shape=jax.ShapeDtypeStruct((N, D), jnp.float32),
    )(agg, HR, b2)


def _fin_body(agg_ref, hr_ref, b_ref, o_ref):
    nb = pl.program_id(0)
    hcat = (jnp.concatenate([agg_ref[0], agg_ref[1]], axis=1)
            + jnp.concatenate([hr_ref[0, 0], hr_ref[1, 0]], axis=1)
            + b_ref[...])
    hb = jnp.maximum(hcat, 0.0)
    part = jnp.sum(hb, axis=0, keepdims=True)

    @pl.when(nb == 0)
    def _():
        o_ref[...] = jnp.zeros_like(o_ref)

    o_ref[...] += part


def _fin(agg, HR, b2):
    return pl.pallas_call(
        _fin_body,
        grid=(N // BN,),
        in_specs=[
            pl.BlockSpec((NC, BN, H), lambda nb: (0, nb, 0)),
            pl.BlockSpec((NC, 1, BN, H), lambda nb: (0, R, nb, 0)),
            pl.BlockSpec((1, D), lambda nb: (0, 0)),
        ],
        out_specs=pl.BlockSpec((1, D), lambda nb: (0, 0)),
        out_shape=jax.ShapeDtypeStruct((1, D), jnp.float32),
    )(agg, HR, b2)


def _sc_scatter(hr_flat, pidx, zrows):
    mesh = plsc.VectorSubcoreMesh(core_axis_name="c", subcore_axis_name="s")

    @functools.partial(
        pl.kernel,
        mesh=mesh,
        out_type=jax.ShapeDtypeStruct((NC, ACC, H), jnp.float32),
        scratch_types=[
            pltpu.VMEM((P_SUB,), jnp.int32),
            pltpu.VMEM((CH,), jnp.int32),
            pltpu.VMEM((CH,), jnp.int32),
            pltpu.VMEM((CH,), jnp.int32),
            pltpu.VMEM((CH,), jnp.int32),
            pltpu.VMEM((2, CH, H), jnp.float32),
            pltpu.VMEM_SHARED((ACC, H), jnp.float32),
            pltpu.SemaphoreType.DMA,
            pltpu.SemaphoreType.DMA,
        ],
    )
    def k(hr_hbm, pidx_hbm, z_hbm, out_hbm, pks, gixb0, gixb1, dstb0, dstb1,
          rows, acc_s, sem0, sem1):
        cid = lax.axis_index("c")
        sid = lax.axis_index("s")
        row0 = sid * ROWS_PER_TILE
        pltpu.sync_copy(z_hbm.at[pl.ds(row0, ROWS_PER_TILE)],
                        acc_s.at[pl.ds(row0, ROWS_PER_TILE)])
        pltpu.sync_copy(pidx_hbm.at[pl.ds(sid * P_SUB, P_SUB)], pks)

        coff = cid * R9N
        gixb = (gixb0, gixb1)
        dstb = (dstb0, dstb1)
        sems = (sem0, sem1)

        def unpack(j, b):
            for i in range(CH // 16):
                p = pks[pl.ds(j * CH + i * 16, 16)]
                sl = pl.ds(i * 16, 16)
                gixb[b][sl] = (p >> DST_BITS) + coff
                dstb[b][sl] = p & (2**DST_BITS - 1)

        def g_start(b):
            pltpu.async_copy(hr_hbm.at[gixb[b]], rows.at[b], sems[b])

        def g_wait(b):
            pltpu.make_async_copy(hr_hbm.at[gixb[b]], rows.at[b],
                                  sems[b]).wait()

        def s_sync(b):
            pltpu.sync_copy(rows.at[b], acc_s.at[dstb[b]], add=True)

        plsc.subcore_barrier()

        unpack(0, 0)
        unpack(1, 1)
        g_start(0)
        g_start(1)

        def body(g, carry):
            j0 = 2 * g
            g_wait(0)
            s_sync(0)

            @pl.when(g < NP - 1)
            def _():
                unpack(j0 + 2, 0)
                g_start(0)

            g_wait(1)
            s_sync(1)

            @pl.when(g < NP - 1)
            def _():
                unpack(j0 + 3, 1)
                g_start(1)

            return carry

        lax.fori_loop(0, NP, body, 0)
        plsc.subcore_barrier()

        @pl.when(cid == 0)
        def _():
            pltpu.sync_copy(acc_s.at[pl.ds(row0, ROWS_PER_TILE)],
                            out_hbm.at[0, pl.ds(row0, ROWS_PER_TILE)])

        @pl.when(cid == 1)
        def _():
            pltpu.sync_copy(acc_s.at[pl.ds(row0, ROWS_PER_TILE)],
                            out_hbm.at[1, pl.ds(row0, ROWS_PER_TILE)])

    return k(hr_flat, pidx, zrows)


def kernel(x, edge_index, edge_type, W0, S0, b0, W1, S1, b1, W2, S2, b2,
           W3, S3, b3):
    src, dst = edge_index[0], edge_index[1]
    gidx = edge_type * N + src
    pad = E_PAD - E
    gidx_p = jnp.concatenate([gidx, jnp.zeros((pad,), jnp.int32)])
    dst_p = jnp.concatenate([dst, jnp.full((pad,), N, jnp.int32)])
    pidx = gidx_p * 2**DST_BITS + dst_p
    zrows = jnp.zeros((ACC, H), jnp.float32)

    layers = ((W0, S0, b0), (W1, S1, b1), (W2, S2, b2), (W3, S3, b3))
    h = x
    agg = HR = None
    blast = None
    for l in range(4):
        W, S, b = layers[l]
        W9 = jnp.concatenate([W, S.reshape(1, D, D)], axis=0)
        HR = _hrk(h, W9)
        agg = _sc_scatter(HR.reshape(NC * R9N, H), pidx, zrows)
        blast = b.reshape(1, D)
        if l < 3:
            h = _prep(agg, HR, blast)
    out = _fin(agg, HR, blast)
    return out.reshape(1, 1, D)
